# baseline (speedup 1.0000x reference)
import jax
import jax.numpy as jnp
from jax.experimental import pallas as pl
from jax.experimental.pallas import tpu as pltpu

_D = 128
_NC = 4
_ROWS_BIG = 100000 // _NC
_ROWS_X = 50000 // _NC


def _copy_body(x_h, a_h, f_h, ao_h, fo_h, xo_h, *sems):
    copies = []
    for c in range(_NC):
        copies.append(pltpu.make_async_copy(
            a_h.at[pl.ds(c * _ROWS_BIG, _ROWS_BIG)],
            ao_h.at[pl.ds(c * _ROWS_BIG, _ROWS_BIG)], sems[3 * c]))
        copies.append(pltpu.make_async_copy(
            f_h.at[pl.ds(c * _ROWS_BIG, _ROWS_BIG)],
            fo_h.at[pl.ds(c * _ROWS_BIG, _ROWS_BIG)], sems[3 * c + 1]))
        copies.append(pltpu.make_async_copy(
            x_h.at[pl.ds(c * _ROWS_X, _ROWS_X)],
            xo_h.at[pl.ds(c * _ROWS_X, _ROWS_X)], sems[3 * c + 2]))
    for cp in copies:
        cp.start()
    for cp in copies:
        cp.wait()


def kernel(x, author_embed, field_embed):
    out = pl.pallas_call(
        _copy_body,
        in_specs=[pl.BlockSpec(memory_space=pl.ANY)] * 3,
        out_specs=[pl.BlockSpec(memory_space=pl.ANY)] * 3,
        out_shape=[
            jax.ShapeDtypeStruct(author_embed.shape, author_embed.dtype),
            jax.ShapeDtypeStruct(field_embed.shape, field_embed.dtype),
            jax.ShapeDtypeStruct(x.shape, x.dtype),
        ],
        scratch_shapes=[pltpu.SemaphoreType.DMA for _ in range(3 * _NC)],
    )(x, author_embed, field_embed)
    return (out[0], out[1], out[2])


# speedup vs baseline: 47.8384x; 47.8384x over previous
import jax
import jax.numpy as jnp
from jax.experimental import pallas as pl
from jax.experimental.pallas import tpu as pltpu

_GRID = 25
_ROWS_BIG = 100000 // _GRID
_ROWS_X = 50000 // _GRID
_D = 128


def _copy3_body(x_ref, a_ref, f_ref, ao_ref, fo_ref, xo_ref):
    ao_ref[...] = a_ref[...]
    fo_ref[...] = f_ref[...]
    xo_ref[...] = x_ref[...]


def kernel(x, author_embed, field_embed):
    out = pl.pallas_call(
        _copy3_body,
        grid=(_GRID,),
        in_specs=[
            pl.BlockSpec((_ROWS_X, _D), lambda i: (i, 0)),
            pl.BlockSpec((_ROWS_BIG, _D), lambda i: (i, 0)),
            pl.BlockSpec((_ROWS_BIG, _D), lambda i: (i, 0)),
        ],
        out_specs=[
            pl.BlockSpec((_ROWS_BIG, _D), lambda i: (i, 0)),
            pl.BlockSpec((_ROWS_BIG, _D), lambda i: (i, 0)),
            pl.BlockSpec((_ROWS_X, _D), lambda i: (i, 0)),
        ],
        out_shape=[
            jax.ShapeDtypeStruct(author_embed.shape, author_embed.dtype),
            jax.ShapeDtypeStruct(field_embed.shape, field_embed.dtype),
            jax.ShapeDtypeStruct(x.shape, x.dtype),
        ],
    )(x, author_embed, field_embed)
    return (out[0], out[1], out[2])
